# baseline (speedup 1.0000x reference)
import jax
import jax.numpy as jnp
from jax.experimental import pallas as pl
from jax.experimental.pallas import tpu as pltpu

N = 8192
H = 128
W = 128
TANFOVX = 0.5
TANFOVY = 0.5
SCALE_MOD = 1.0
FX = W / (2.0 * TANFOVX)
FY = H / (2.0 * TANFOVY)

NSTRIP = 16
SH = H // NSTRIP
KDUP = 3
LEN = KDUP * N
NSEG = NSTRIP + 1
BIGKEY = NSTRIP
DUMKEY = NSTRIP + 1
LEN_PAD = LEN + (SH - 1) * NSEG + 1


def _cov3d(scales, rotations):
    q = rotations / jnp.linalg.norm(rotations, axis=1, keepdims=True)
    r, x, y, z = q[:, 0], q[:, 1], q[:, 2], q[:, 3]
    R = jnp.stack([1 - 2 * (y * y + z * z), 2 * (x * y - r * z), 2 * (x * z + r * y),
                   2 * (x * y + r * z), 1 - 2 * (x * x + z * z), 2 * (y * z - r * x),
                   2 * (x * z - r * y), 2 * (y * z + r * x), 1 - 2 * (x * x + y * y)],
                  axis=1).reshape(-1, 3, 3)
    M = R * (scales * SCALE_MOD)[:, None, :]
    return M @ jnp.swapaxes(M, 1, 2)


def _project(means3D, opacities, scales, rotations):
    t = means3D
    depth = t[:, 2]
    visible = depth > 0.2
    tz = jnp.where(visible, depth, 1.0)
    limx = 1.3 * TANFOVX
    limy = 1.3 * TANFOVY
    tx = jnp.clip(t[:, 0] / tz, -limx, limx) * tz
    ty = jnp.clip(t[:, 1] / tz, -limy, limy) * tz
    cov3d = _cov3d(scales, rotations)
    Nn = t.shape[0]
    J = jnp.zeros((Nn, 2, 3), dtype=jnp.float32)
    J = J.at[:, 0, 0].set(FX / tz).at[:, 0, 2].set(-FX * tx / (tz * tz))
    J = J.at[:, 1, 1].set(FY / tz).at[:, 1, 2].set(-FY * ty / (tz * tz))
    cov2d = jnp.einsum('nij,njk,nlk->nil', J, cov3d, J)
    a = cov2d[:, 0, 0] + 0.3
    c_ = cov2d[:, 1, 1] + 0.3
    b = cov2d[:, 0, 1]
    det = a * c_ - b * b
    det_ok = det > 0
    det_s = jnp.where(det_ok, det, 1.0)
    conic_a = c_ / det_s
    conic_b = -b / det_s
    conic_c = a / det_s
    px = (t[:, 0] / (tz * TANFOVX) + 1.0) * 0.5 * W - 0.5
    py = (t[:, 1] / (tz * TANFOVY) + 1.0) * 0.5 * H - 0.5
    mid = 0.5 * (a + c_)
    lam1 = mid + jnp.sqrt(jnp.maximum(mid * mid - det_s, 0.1))
    radii = jnp.where(visible & det_ok, jnp.ceil(3.0 * jnp.sqrt(lam1)), 0.0).astype(jnp.int32)
    valid = visible & det_ok & (radii > 0)
    op = jnp.where(valid, opacities[:, 0], 0.0)
    return px, py, conic_a, conic_b, conic_c, op, depth, lam1, radii, valid


def _composite(T, o0, o1, o2, xs, ys, px, py, ca, cb, cc, op, d, f2):
    dx = xs - px
    dy = ys - py
    power = -0.5 * (ca * dx * dx + cc * dy * dy) - cb * dx * dy
    alpha = jnp.minimum(0.99, op * jnp.exp(power))
    alpha = jnp.where((power <= 0.0) & (alpha >= 1.0 / 255.0), alpha, 0.0)
    w = T * alpha
    return (T * (1.0 - alpha), o0 + w * d, o1 + w, o2 + w * f2)


def _render_body(astart_ref, counts_ref, starts_ref, glist_ref, par_ref,
                 parb_ref, color_ref):
    s = pl.program_id(0)
    ys = (jax.lax.broadcasted_iota(jnp.int32, (SH, W), 0) + s * SH).astype(jnp.float32)
    xs = jax.lax.broadcasted_iota(jnp.int32, (SH, W), 1).astype(jnp.float32)
    inf = jnp.float32(jnp.inf)

    init = (jnp.ones((SH, W), jnp.float32),
            jnp.zeros((SH, W), jnp.float32),
            jnp.zeros((SH, W), jnp.float32),
            jnp.zeros((SH, W), jnp.float32))

    def fast_path(_):
        base0 = astart_ref[s]
        nch = (counts_ref[s] + 7) // 8
        xs128 = jax.lax.broadcasted_iota(jnp.int32, (8, W), 1).astype(jnp.float32)
        ones1 = jnp.ones((1, W), jnp.float32)
        ones2 = jnp.ones((2, W), jnp.float32)
        ones4 = jnp.ones((4, W), jnp.float32)
        sy = s * SH

        def chunk_body(k, carry):
            Ts, o0s, o1s, o2s = carry
            base = pl.multiple_of(base0 + k * 8, 8)
            ch = parb_ref[pl.ds(base, 8), :]
            gpx = ch[:, 0:1]
            gpy = ch[:, 1:2]
            gca = ch[:, 2:3]
            gcb = ch[:, 3:4]
            gcc = ch[:, 4:5]
            gop = ch[:, 5:6]
            gd = ch[:, 6:7]
            gf2 = ch[:, 7:8]
            dx = xs128 - gpx
            adx2 = (-0.5 * gca) * (dx * dx)
            negcb = -gcb
            nT, n0, n1, n2 = [], [], [], []
            for r in range(SH):
                yr = (sy + r).astype(jnp.float32)
                dy = yr - gpy
                bt = negcb * dy
                ct = (-0.5 * gcc) * (dy * dy)
                power = adx2 + bt * dx + ct
                al = jnp.minimum(0.99, gop * jnp.exp(power))
                al = jnp.where((power <= 0.0) & (al >= 1.0 / 255.0), al, 0.0)
                v = 1.0 - al
                v = v * jnp.concatenate([ones1, v[:-1]], axis=0)
                v = v * jnp.concatenate([ones2, v[:-2]], axis=0)
                v = v * jnp.concatenate([ones4, v[:-4]], axis=0)
                excl = jnp.concatenate([ones1, v[:-1]], axis=0)
                w = (Ts[r] * excl) * al
                n0.append(o0s[r] + jnp.sum(w * gd, axis=0, keepdims=True))
                n1.append(o1s[r] + jnp.sum(w, axis=0, keepdims=True))
                n2.append(o2s[r] + jnp.sum(w * gf2, axis=0, keepdims=True))
                nT.append(Ts[r] * v[SH - 1:SH])
            return (tuple(nT), tuple(n0), tuple(n1), tuple(n2))

        rinit = (tuple(jnp.ones((1, W), jnp.float32) for _ in range(SH)),
                 tuple(jnp.zeros((1, W), jnp.float32) for _ in range(SH)),
                 tuple(jnp.zeros((1, W), jnp.float32) for _ in range(SH)),
                 tuple(jnp.zeros((1, W), jnp.float32) for _ in range(SH)))
        Ts, o0s, o1s, o2s = jax.lax.fori_loop(0, nch, chunk_body, rinit)
        return (jnp.concatenate(Ts, axis=0), jnp.concatenate(o0s, axis=0),
                jnp.concatenate(o1s, axis=0), jnp.concatenate(o2s, axis=0))

    def slow_path(_):
        a0 = starts_ref[s]
        na = starts_ref[s + 1] - a0
        b0 = starts_ref[BIGKEY]
        nb = starts_ref[BIGKEY + 1] - b0

        def body(k, carry):
            T, o0, o1, o2, ia, ib = carry
            ga = glist_ref[jnp.minimum(a0 + ia, LEN - 1)]
            gb = glist_ref[jnp.minimum(b0 + ib, LEN - 1)]
            da = jnp.where(ia < na, par_ref[6, ga], inf)
            db = jnp.where(ib < nb, par_ref[6, gb], inf)
            use_a = (da < db) | ((da == db) & (ga <= gb))
            g = jnp.where(use_a, ga, gb)
            ia = jnp.where(use_a, ia + 1, ia)
            ib = jnp.where(use_a, ib, ib + 1)
            T, o0, o1, o2 = _composite(
                T, o0, o1, o2, xs, ys,
                par_ref[0, g], par_ref[1, g], par_ref[2, g], par_ref[3, g],
                par_ref[4, g], par_ref[5, g], par_ref[6, g],
                1.0 / (1.0 + jnp.maximum(par_ref[6, g], 0.0)))
            return (T, o0, o1, o2, ia, ib)

        out = jax.lax.fori_loop(0, na + nb, body, init + (jnp.int32(0), jnp.int32(0)))
        return out[:4]

    nbig = counts_ref[BIGKEY]
    T, o0, o1, o2 = jax.lax.cond(nbig == 0, fast_path, slow_path, 0)
    color_ref[0] = o0
    color_ref[1] = o1
    color_ref[2] = o2


def kernel(means3D, means2D, opacities, scales, rotations):
    px, py, ca, cb, cc, op, depth, lam1, radii, valid = _project(
        means3D, opacities, scales, rotations)
    sortkey = jnp.where(valid, depth, jnp.inf)
    order = jnp.argsort(sortkey)
    pars = jnp.stack([px[order], py[order], ca[order], cb[order], cc[order],
                      op[order], depth[order]])

    op_s = pars[5]
    py_s = pars[1]
    lam1_s = lam1[order]
    r_cut = jnp.sqrt(jnp.maximum(2.0 * lam1_s * jnp.log(255.0 * op_s), 0.0)) + 1.0
    never = op_s * 255.0 <= 1.0
    y0 = py_s - r_cut
    y1 = py_s + r_cut
    s0 = jnp.floor(y0 / SH).astype(jnp.int32)
    s1 = jnp.floor(y1 / SH).astype(jnp.int32)
    s0c = jnp.maximum(s0, 0)
    s1c = jnp.minimum(s1, NSTRIP - 1)
    never = never | (s1c < s0c)
    big = (s1c - s0c + 1) > KDUP
    keys = []
    for o in range(KDUP):
        k_o = jnp.where(s0c + o <= s1c, s0c + o, DUMKEY)
        k_o = jnp.where(big, BIGKEY if o == 0 else DUMKEY, k_o)
        k_o = jnp.where(never, DUMKEY, k_o)
        keys.append(k_o)
    keys_flat = jnp.stack(keys, axis=1).reshape(LEN)
    bperm = jnp.argsort(keys_flat, stable=True)
    glist = (bperm // KDUP).astype(jnp.int32)
    keys_sorted = keys_flat[bperm]
    starts = jnp.searchsorted(keys_sorted, jnp.arange(DUMKEY + 1),
                              side='left').astype(jnp.int32)

    counts = starts[1:NSEG + 1] - starts[:NSEG]
    cnt_pad = ((counts + (SH - 1)) // 8) * 8
    astart = jnp.concatenate([jnp.zeros((1,), jnp.int32),
                              jnp.cumsum(cnt_pad).astype(jnp.int32)])
    j = jnp.arange(LEN_PAD, dtype=jnp.int32)
    seg = jnp.minimum(
        jnp.searchsorted(astart[1:], j, side='right').astype(jnp.int32),
        NSEG - 1)
    off = j - astart[seg]
    val = off < counts[seg]
    e = jnp.clip(starts[seg] + off, 0, LEN - 1)
    g_j = jnp.where(val, glist[e], N)
    f2_s = 1.0 / (1.0 + jnp.maximum(pars[6], 0.0))
    pars8 = jnp.concatenate(
        [jnp.transpose(pars), f2_s[:, None]], axis=1)
    pars8 = jnp.concatenate(
        [pars8, jnp.zeros((1, 8), jnp.float32)], axis=0)
    par_binned = pars8[g_j]

    color = pl.pallas_call(
        _render_body,
        grid=(NSTRIP,),
        in_specs=[pl.BlockSpec(memory_space=pltpu.SMEM),
                  pl.BlockSpec(memory_space=pltpu.SMEM),
                  pl.BlockSpec(memory_space=pltpu.SMEM),
                  pl.BlockSpec(memory_space=pltpu.SMEM),
                  pl.BlockSpec(memory_space=pltpu.SMEM),
                  pl.BlockSpec(memory_space=pltpu.VMEM)],
        out_specs=pl.BlockSpec((3, SH, W), lambda i: (0, i, 0)),
        out_shape=jax.ShapeDtypeStruct((3, H, W), jnp.float32),
    )(astart, counts, starts, glist, pars, par_binned)
    return color, radii


# speedup vs baseline: 1.9853x; 1.9853x over previous
import jax
import jax.numpy as jnp
from jax.experimental import pallas as pl
from jax.experimental.pallas import tpu as pltpu

N = 8192
H = 128
W = 128
TANFOVX = 0.5
TANFOVY = 0.5
SCALE_MOD = 1.0
FX = W / (2.0 * TANFOVX)
FY = H / (2.0 * TANFOVY)


def _cov3d(scales, rotations):
    q = rotations / jnp.linalg.norm(rotations, axis=1, keepdims=True)
    r, x, y, z = q[:, 0], q[:, 1], q[:, 2], q[:, 3]
    R = jnp.stack([1 - 2 * (y * y + z * z), 2 * (x * y - r * z), 2 * (x * z + r * y),
                   2 * (x * y + r * z), 1 - 2 * (x * x + z * z), 2 * (y * z - r * x),
                   2 * (x * z - r * y), 2 * (y * z + r * x), 1 - 2 * (x * x + y * y)],
                  axis=1).reshape(-1, 3, 3)
    M = R * (scales * SCALE_MOD)[:, None, :]
    return M @ jnp.swapaxes(M, 1, 2)


def _project(means3D, opacities, scales, rotations):
    t = means3D
    depth = t[:, 2]
    visible = depth > 0.2
    tz = jnp.where(visible, depth, 1.0)
    limx = 1.3 * TANFOVX
    limy = 1.3 * TANFOVY
    tx = jnp.clip(t[:, 0] / tz, -limx, limx) * tz
    ty = jnp.clip(t[:, 1] / tz, -limy, limy) * tz
    cov3d = _cov3d(scales, rotations)
    Nn = t.shape[0]
    J = jnp.zeros((Nn, 2, 3), dtype=jnp.float32)
    J = J.at[:, 0, 0].set(FX / tz).at[:, 0, 2].set(-FX * tx / (tz * tz))
    J = J.at[:, 1, 1].set(FY / tz).at[:, 1, 2].set(-FY * ty / (tz * tz))
    cov2d = jnp.einsum('nij,njk,nlk->nil', J, cov3d, J)
    a = cov2d[:, 0, 0] + 0.3
    c_ = cov2d[:, 1, 1] + 0.3
    b = cov2d[:, 0, 1]
    det = a * c_ - b * b
    det_ok = det > 0
    det_s = jnp.where(det_ok, det, 1.0)
    conic_a = c_ / det_s
    conic_b = -b / det_s
    conic_c = a / det_s
    px = (t[:, 0] / (tz * TANFOVX) + 1.0) * 0.5 * W - 0.5
    py = (t[:, 1] / (tz * TANFOVY) + 1.0) * 0.5 * H - 0.5
    mid = 0.5 * (a + c_)
    lam1 = mid + jnp.sqrt(jnp.maximum(mid * mid - det_s, 0.1))
    radii = jnp.where(visible & det_ok, jnp.ceil(3.0 * jnp.sqrt(lam1)), 0.0).astype(jnp.int32)
    valid = visible & det_ok & (radii > 0)
    op = jnp.where(valid, opacities[:, 0], 0.0)
    return px, py, conic_a, conic_b, conic_c, op, depth, radii, valid


def _render_body(par_ref, color_ref):
    ys = jax.lax.broadcasted_iota(jnp.int32, (H, W), 0).astype(jnp.float32)
    xs = jax.lax.broadcasted_iota(jnp.int32, (H, W), 1).astype(jnp.float32)

    def body(g, carry):
        T, o0, o1, o2 = carry
        px = par_ref[0, g]
        py = par_ref[1, g]
        A = par_ref[2, g]
        B = par_ref[3, g]
        C = par_ref[4, g]
        op = par_ref[5, g]
        d = par_ref[6, g]
        f2 = par_ref[7, g]
        dx = xs - px
        dy = ys - py
        power = dx * (A * dx + B * dy) + C * (dy * dy)
        alpha = jnp.minimum(0.99, op * jnp.exp(power))
        alpha = jnp.where((power <= 0.0) & (alpha >= 1.0 / 255.0), alpha, 0.0)
        w = T * alpha
        return (T * (1.0 - alpha), o0 + w * d, o1 + w, o2 + w * f2)

    zero = jnp.zeros((H, W), jnp.float32)
    ones = jnp.ones((H, W), jnp.float32)
    T, o0, o1, o2 = jax.lax.fori_loop(0, N, body, (ones, zero, zero, zero))
    color_ref[0] = o0
    color_ref[1] = o1
    color_ref[2] = o2


def kernel(means3D, means2D, opacities, scales, rotations):
    px, py, ca, cb, cc, op, depth, radii, valid = _project(
        means3D, opacities, scales, rotations)
    sortkey = jnp.where(valid, depth, jnp.inf)
    order = jnp.argsort(sortkey)
    f2 = 1.0 / (1.0 + jnp.maximum(depth, 0.0))
    pars = jnp.stack([px[order], py[order], (-0.5 * ca)[order], (-cb)[order],
                      (-0.5 * cc)[order], op[order], depth[order],
                      f2[order]])
    color = pl.pallas_call(
        _render_body,
        in_specs=[pl.BlockSpec(memory_space=pltpu.SMEM)],
        out_shape=jax.ShapeDtypeStruct((3, H, W), jnp.float32),
    )(pars)
    return color, radii


# speedup vs baseline: 2.4139x; 1.2159x over previous
import jax
import jax.numpy as jnp
from jax.experimental import pallas as pl
from jax.experimental.pallas import tpu as pltpu

N = 8192
H = 128
W = 128
TANFOVX = 0.5
TANFOVY = 0.5
SCALE_MOD = 1.0
FX = W / (2.0 * TANFOVX)
FY = H / (2.0 * TANFOVY)

NQ = 8
QH = H // NQ


def _cov3d(scales, rotations):
    q = rotations / jnp.linalg.norm(rotations, axis=1, keepdims=True)
    r, x, y, z = q[:, 0], q[:, 1], q[:, 2], q[:, 3]
    R = jnp.stack([1 - 2 * (y * y + z * z), 2 * (x * y - r * z), 2 * (x * z + r * y),
                   2 * (x * y + r * z), 1 - 2 * (x * x + z * z), 2 * (y * z - r * x),
                   2 * (x * z - r * y), 2 * (y * z + r * x), 1 - 2 * (x * x + y * y)],
                  axis=1).reshape(-1, 3, 3)
    M = R * (scales * SCALE_MOD)[:, None, :]
    return M @ jnp.swapaxes(M, 1, 2)


def _project(means3D, opacities, scales, rotations):
    t = means3D
    depth = t[:, 2]
    visible = depth > 0.2
    tz = jnp.where(visible, depth, 1.0)
    limx = 1.3 * TANFOVX
    limy = 1.3 * TANFOVY
    tx = jnp.clip(t[:, 0] / tz, -limx, limx) * tz
    ty = jnp.clip(t[:, 1] / tz, -limy, limy) * tz
    cov3d = _cov3d(scales, rotations)
    Nn = t.shape[0]
    J = jnp.zeros((Nn, 2, 3), dtype=jnp.float32)
    J = J.at[:, 0, 0].set(FX / tz).at[:, 0, 2].set(-FX * tx / (tz * tz))
    J = J.at[:, 1, 1].set(FY / tz).at[:, 1, 2].set(-FY * ty / (tz * tz))
    cov2d = jnp.einsum('nij,njk,nlk->nil', J, cov3d, J)
    a = cov2d[:, 0, 0] + 0.3
    c_ = cov2d[:, 1, 1] + 0.3
    b = cov2d[:, 0, 1]
    det = a * c_ - b * b
    det_ok = det > 0
    det_s = jnp.where(det_ok, det, 1.0)
    conic_a = c_ / det_s
    conic_b = -b / det_s
    conic_c = a / det_s
    px = (t[:, 0] / (tz * TANFOVX) + 1.0) * 0.5 * W - 0.5
    py = (t[:, 1] / (tz * TANFOVY) + 1.0) * 0.5 * H - 0.5
    mid = 0.5 * (a + c_)
    lam1 = mid + jnp.sqrt(jnp.maximum(mid * mid - det_s, 0.1))
    radii = jnp.where(visible & det_ok, jnp.ceil(3.0 * jnp.sqrt(lam1)), 0.0).astype(jnp.int32)
    valid = visible & det_ok & (radii > 0)
    op = jnp.where(valid, opacities[:, 0], 0.0)
    return px, py, conic_a, conic_b, conic_c, op, depth, lam1, radii, valid


def _render_body(par_ref, qb_ref, color_ref, T_ref, o0_ref, o1_ref, o2_ref):
    T_ref[...] = jnp.ones((H, W), jnp.float32)
    o0_ref[...] = jnp.zeros((H, W), jnp.float32)
    o1_ref[...] = jnp.zeros((H, W), jnp.float32)
    o2_ref[...] = jnp.zeros((H, W), jnp.float32)

    def body(g, _):
        px = par_ref[0, g]
        py = par_ref[1, g]
        A = par_ref[2, g]
        B = par_ref[3, g]
        C = par_ref[4, g]
        op = par_ref[5, g]
        d = par_ref[6, g]
        f2 = par_ref[7, g]
        qlo = qb_ref[0, g]
        qhi = qb_ref[1, g]

        def qstep(r, __):
            base = pl.multiple_of(r * QH, QH)
            ys = (jax.lax.broadcasted_iota(jnp.int32, (QH, W), 0) + base
                  ).astype(jnp.float32)
            xs = jax.lax.broadcasted_iota(jnp.int32, (QH, W), 1).astype(jnp.float32)
            dx = xs - px
            dy = ys - py
            power = dx * (A * dx + B * dy) + C * (dy * dy)
            alpha = jnp.minimum(0.99, op * jnp.exp(power))
            alpha = jnp.where((power <= 0.0) & (alpha >= 1.0 / 255.0), alpha, 0.0)
            T = T_ref[pl.ds(base, QH), :]
            w = T * alpha
            o0_ref[pl.ds(base, QH), :] += w * d
            o1_ref[pl.ds(base, QH), :] += w
            o2_ref[pl.ds(base, QH), :] += w * f2
            T_ref[pl.ds(base, QH), :] = T * (1.0 - alpha)
            return 0

        jax.lax.fori_loop(qlo, qhi + 1, qstep, 0)
        return 0

    jax.lax.fori_loop(0, N, body, 0)
    color_ref[0] = o0_ref[...]
    color_ref[1] = o1_ref[...]
    color_ref[2] = o2_ref[...]


def kernel(means3D, means2D, opacities, scales, rotations):
    px, py, ca, cb, cc, op, depth, lam1, radii, valid = _project(
        means3D, opacities, scales, rotations)
    sortkey = jnp.where(valid, depth, jnp.inf)
    order = jnp.argsort(sortkey)
    f2 = 1.0 / (1.0 + jnp.maximum(depth, 0.0))
    pars = jnp.stack([px[order], py[order], (-0.5 * ca)[order], (-cb)[order],
                      (-0.5 * cc)[order], op[order], depth[order],
                      f2[order]])

    op_s = pars[5]
    py_s = pars[1]
    lam1_s = lam1[order]
    r_cut = jnp.sqrt(jnp.maximum(2.0 * lam1_s * jnp.log(255.0 * op_s), 0.0)) + 1.0
    never = op_s * 255.0 <= 1.0
    qlo = jnp.clip(jnp.floor((py_s - r_cut) / QH), 0, NQ - 1).astype(jnp.int32)
    qhi = jnp.clip(jnp.floor((py_s + r_cut) / QH), 0, NQ - 1).astype(jnp.int32)
    offscreen = (py_s + r_cut < 0.0) | (py_s - r_cut > H - 1)
    skip = never | offscreen
    qlo = jnp.where(skip, 1, qlo)
    qhi = jnp.where(skip, 0, qhi)
    qb = jnp.stack([qlo, qhi])

    color = pl.pallas_call(
        _render_body,
        in_specs=[pl.BlockSpec(memory_space=pltpu.SMEM),
                  pl.BlockSpec(memory_space=pltpu.SMEM)],
        out_shape=jax.ShapeDtypeStruct((3, H, W), jnp.float32),
        scratch_shapes=[pltpu.VMEM((H, W), jnp.float32)] * 4,
    )(pars, qb)
    return color, radii
